# baseline (speedup 1.0000x reference)
import functools

import jax
import jax.numpy as jnp
from jax import lax
from jax.experimental import pallas as pl
from jax.experimental.pallas import tpu as pltpu
from jax.experimental.pallas import tpu_sc as plsc


def _pa_body(rf_ref, w_ref, pa_ref):
    rf = rf_ref[...]
    w = w_ref[...]
    s = jnp.sum(rf * w, axis=0, keepdims=True)
    pa_ref[...] = jax.nn.sigmoid(s)


def _h_body(f_ref, cj_ref, h_ref):
    h_ref[...] = f_ref[...] * cj_ref[...]


def _combine_body(p0_ref, p1_ref, ci_ref, o_ref):
    o_ref[...] = (p0_ref[...] + p1_ref[...]) * ci_ref[...]


def _make_sc_kernel(N, E, D, nch, origin, load_acc):
    NC, NS, L = 2, 16, 16
    NW = NC * NS
    C = 80
    EPT = nch * C
    G = C // L
    SR = 80
    NSLAB = N // SR
    KMAX = -(-NSLAB // NS)

    mesh = plsc.VectorSubcoreMesh(core_axis_name="c", subcore_axis_name="s")

    def _impl(h_hbm, pa_hbm, src_hbm, dst_hbm, pin_hbm, out_hbm,
              srcc, pac, dstc, rows, acc, semg, semsrc, sempa, semd, sems):
        cid = lax.axis_index("c")
        sid = lax.axis_index("s")
        wid = sid * NC + cid
        ebase = origin + wid * EPT
        pbase = wid * EPT

        def _slabs(fn):
            for k in range(KMAX):
                slab = sid + NS * k
                if (k + 1) * NS <= NSLAB:
                    fn(slab)
                else:
                    @pl.when(slab < NSLAB)
                    def _():
                        fn(slab)

        if load_acc:
            def _load(slab):
                pltpu.sync_copy(
                    pin_hbm.at[pl.ds(cid * N + slab * SR, SR)], rows[0])
                pltpu.sync_copy(rows[0], acc.at[pl.ds(slab * SR, SR)])
            _slabs(_load)
        else:
            def _zr(r, carry):
                for d in range(D // L):
                    rows[0][r, pl.ds(d * L, L)] = jnp.zeros((L,), jnp.float32)
                return carry
            lax.fori_loop(0, C, _zr, 0)

            def _zero(slab):
                pltpu.sync_copy(rows[0], acc.at[pl.ds(slab * SR, SR)])
            _slabs(_zero)

        plsc.subcore_barrier()

        def _start_gather(b):
            pltpu.async_copy(h_hbm.at[srcc[b]], rows[b], semg[b])

        def _wait_gather(b):
            pltpu.make_async_copy(h_hbm.at[srcc[b]], rows[b], semg[b]).wait()

        def _start_srcpa(ch, b):
            pltpu.async_copy(
                src_hbm.at[pl.ds(ebase + ch * C, C)], srcc[b], semsrc[b])
            pltpu.async_copy(
                pa_hbm.at[pl.ds(pbase + ch * C, C)], pac[b], sempa[b])

        def _wait_srcpa(ch, b):
            pltpu.make_async_copy(
                src_hbm.at[pl.ds(ebase + ch * C, C)], srcc[b], semsrc[b]).wait()
            pltpu.make_async_copy(
                pa_hbm.at[pl.ds(pbase + ch * C, C)], pac[b], sempa[b]).wait()

        def _start_dst(ch, b):
            pltpu.async_copy(
                dst_hbm.at[pl.ds(ebase + ch * C, C)], dstc[b], semd[b])

        def _wait_dst(ch, b):
            pltpu.make_async_copy(
                dst_hbm.at[pl.ds(ebase + ch * C, C)], dstc[b], semd[b]).wait()

        def _scale(ch, b):
            def _g(g, carry):
                f = pac[b][pl.ds(g * L, L)]
                for e in range(L):
                    s_vec = jnp.full((L,), f[e], jnp.float32)
                    row = g * L + e
                    for d in range(D // L):
                        rows[b][row, pl.ds(d * L, L)] = (
                            rows[b][row, pl.ds(d * L, L)] * s_vec)
                return carry
            lax.fori_loop(0, G, _g, 0)

        def _start_scatter(b):
            pltpu.async_copy(rows[b], acc.at[dstc[b]], sems[b], add=True)

        def _wait_scatter(b):
            pltpu.make_async_copy(rows[b], acc.at[dstc[b]], sems[b]).wait()

        def _clamp(ch):
            return jnp.minimum(ch, nch - 1)

        def _sub(ch, b):
            b2 = (b + 2) % 4
            b3 = (b + 3) % 4
            _wait_scatter(b2)
            _start_dst(_clamp(ch + 2), b2)
            _start_srcpa(_clamp(ch + 3), b3)
            _wait_srcpa(_clamp(ch + 2), b2)
            _start_gather(b2)
            _wait_gather(b)
            _scale(ch, b)
            _wait_dst(ch, b)
            _start_scatter(b)

        _start_srcpa(0, 0)
        _start_srcpa(1, 1)
        _start_srcpa(2, 2)
        _start_dst(0, 0)
        _start_dst(1, 1)
        _wait_srcpa(0, 0)
        _start_gather(0)
        _wait_srcpa(1, 1)
        _start_gather(1)
        _start_dst(2, 2)
        _start_srcpa(3, 3)
        _wait_srcpa(2, 2)
        _start_gather(2)
        _wait_gather(0)
        _scale(0, 0)
        _wait_dst(0, 0)
        _start_scatter(0)
        _start_dst(3, 3)
        _start_srcpa(4, 0)
        _wait_srcpa(3, 3)
        _start_gather(3)
        _wait_gather(1)
        _scale(1, 1)
        _wait_dst(1, 1)
        _start_scatter(1)

        NT = (nch - 2) // 4

        def _quad(t, carry):
            ch = 4 * t + 2
            _sub(ch, 2)
            _sub(ch + 1, 3)
            _sub(ch + 2, 0)
            _sub(ch + 3, 1)
            return carry
        lax.fori_loop(0, NT, _quad, 0)
        for ch in range(2 + 4 * NT, nch):
            _sub(ch, ch % 4)

        _wait_scatter((nch - 2) % 4)
        _wait_scatter((nch - 1) % 4)
        _wait_gather(nch % 4)
        _wait_gather((nch + 1) % 4)
        _wait_srcpa(nch - 1, (nch + 2) % 4)
        _wait_dst(nch - 1, nch % 4)
        _wait_dst(nch - 1, (nch + 1) % 4)

        plsc.subcore_barrier()

        def _dump(slab):
            pltpu.sync_copy(acc.at[pl.ds(slab * SR, SR)], rows[0])
            pltpu.sync_copy(
                rows[0], out_hbm.at[pl.ds(cid * N + slab * SR, SR)])
        _slabs(_dump)

    kw = dict(
        out_type=jax.ShapeDtypeStruct((2 * N, D), jnp.float32),
        mesh=mesh,
        compiler_params=pltpu.CompilerParams(needs_layout_passes=False),
        scratch_types=[
            [pltpu.VMEM((C,), jnp.int32) for _ in range(4)],
            [pltpu.VMEM((C,), jnp.float32) for _ in range(4)],
            [pltpu.VMEM((C,), jnp.int32) for _ in range(4)],
            [pltpu.VMEM((C, D), jnp.float32) for _ in range(4)],
            pltpu.VMEM_SHARED((N, D), jnp.float32),
            [pltpu.SemaphoreType.DMA for _ in range(4)],
            [pltpu.SemaphoreType.DMA for _ in range(4)],
            [pltpu.SemaphoreType.DMA for _ in range(4)],
            [pltpu.SemaphoreType.DMA for _ in range(4)],
            [pltpu.SemaphoreType.DMA for _ in range(4)],
        ],
    )

    if load_acc:
        @functools.partial(pl.kernel, **kw)
        def sck(h_hbm, pa_hbm, src_hbm, dst_hbm, pin_hbm, out_hbm,
                srcc, pac, dstc, rows, acc, semg, semsrc, sempa, semd, sems):
            _impl(h_hbm, pa_hbm, src_hbm, dst_hbm, pin_hbm, out_hbm,
                  srcc, pac, dstc, rows, acc, semg, semsrc, sempa, semd, sems)
    else:
        @functools.partial(pl.kernel, **kw)
        def sck(h_hbm, pa_hbm, src_hbm, dst_hbm, out_hbm,
                srcc, pac, dstc, rows, acc, semg, semsrc, sempa, semd, sems):
            _impl(h_hbm, pa_hbm, src_hbm, dst_hbm, None, out_hbm,
                  srcc, pac, dstc, rows, acc, semg, semsrc, sempa, semd, sems)
    return sck


def _pa_slice(rfT, prob_wT, E_k, block_off):
    RD = rfT.shape[0]
    BE = 6400
    return pl.pallas_call(
        _pa_body,
        grid=(E_k // BE,),
        in_specs=[
            pl.BlockSpec((RD, BE), lambda i: (0, i + block_off)),
            pl.BlockSpec((RD, 1), lambda i: (0, 0)),
        ],
        out_specs=pl.BlockSpec((1, BE), lambda i: (0, i)),
        out_shape=jax.ShapeDtypeStruct((1, E_k), jnp.float32),
    )(rfT, prob_wT)


def kernel(feat, cj, ci, review_feat, prob_w, edge_index):
    N, D = feat.shape
    E, RD = review_feat.shape

    NW, C = 32, 80
    NCH0 = 30
    E0 = NW * C * NCH0
    NCH1 = (E - E0) // (NW * C)
    E1 = E - E0
    BE = 6400

    rfT = review_feat.T
    wT = prob_w.T
    pa0 = _pa_slice(rfT, wT, E0, 0)
    pa1 = _pa_slice(rfT, wT, E1, E0 // BE)

    BN = 2000
    h = pl.pallas_call(
        _h_body,
        grid=(N // BN,),
        in_specs=[
            pl.BlockSpec((BN, D), lambda i: (i, 0)),
            pl.BlockSpec((BN, 1), lambda i: (i, 0)),
        ],
        out_specs=pl.BlockSpec((BN, D), lambda i: (i, 0)),
        out_shape=jax.ShapeDtypeStruct((N, D), jnp.float32),
    )(feat, cj)

    src = edge_index[0]
    dst = edge_index[1]
    sck0 = _make_sc_kernel(N, E, D, NCH0, 0, False)
    sck1 = _make_sc_kernel(N, E, D, NCH1, E0, True)
    partial0 = sck0(h, pa0.reshape(E0), src, dst)
    partial = sck1(h, pa1.reshape(E1), src, dst, partial0)

    NB = N // BN
    out = pl.pallas_call(
        _combine_body,
        grid=(NB,),
        in_specs=[
            pl.BlockSpec((BN, D), lambda i: (i, 0)),
            pl.BlockSpec((BN, D), lambda i: (i + NB, 0)),
            pl.BlockSpec((BN, 1), lambda i: (i, 0)),
        ],
        out_specs=pl.BlockSpec((BN, D), lambda i: (i, 0)),
        out_shape=jax.ShapeDtypeStruct((N, D), jnp.float32),
    )(partial, partial, ci)
    return out


# speedup vs baseline: 1.0284x; 1.0284x over previous
import functools

import jax
import jax.numpy as jnp
from jax import lax
from jax.experimental import pallas as pl
from jax.experimental.pallas import tpu as pltpu
from jax.experimental.pallas import tpu_sc as plsc


def _pa_body(rf_ref, w_ref, pa_ref):
    rf = rf_ref[...]
    w = w_ref[...]
    s = jnp.sum(rf * w, axis=0, keepdims=True)
    pa_ref[...] = jax.nn.sigmoid(s)


def _h_body(f_ref, cj_ref, h_ref):
    h_ref[...] = f_ref[...] * cj_ref[...]


def _combine_body(p0_ref, p1_ref, ci_ref, o_ref):
    o_ref[...] = (p0_ref[...] + p1_ref[...]) * ci_ref[...]


def _make_sc_kernel(N, E, D, nch, origin, load_acc):
    NC, NS, L = 2, 16, 16
    NW = NC * NS
    C = 80
    EPT = nch * C
    G = C // L
    SR = 80
    NSLAB = N // SR
    KMAX = -(-NSLAB // NS)

    mesh = plsc.VectorSubcoreMesh(core_axis_name="c", subcore_axis_name="s")

    def _impl(h_hbm, pa_hbm, src_hbm, dst_hbm, pin_hbm, out_hbm,
              srcc, pac, dstc, rows, acc, semg, semsrc, sempa, semd, sems):
        cid = lax.axis_index("c")
        sid = lax.axis_index("s")
        wid = sid * NC + cid
        ebase = origin + wid * EPT
        pbase = wid * EPT

        def _slabs(fn):
            for k in range(KMAX):
                slab = sid + NS * k
                if (k + 1) * NS <= NSLAB:
                    fn(slab)
                else:
                    @pl.when(slab < NSLAB)
                    def _():
                        fn(slab)

        if load_acc:
            def _load(slab):
                pltpu.sync_copy(
                    pin_hbm.at[pl.ds(cid * N + slab * SR, SR)], rows[0])
                pltpu.sync_copy(rows[0], acc.at[pl.ds(slab * SR, SR)])
            _slabs(_load)
        else:
            def _zr(r, carry):
                for d in range(D // L):
                    rows[0][r, pl.ds(d * L, L)] = jnp.zeros((L,), jnp.float32)
                return carry
            lax.fori_loop(0, C, _zr, 0)

            def _zero(slab):
                pltpu.sync_copy(rows[0], acc.at[pl.ds(slab * SR, SR)])
            _slabs(_zero)

        plsc.subcore_barrier()

        def _start_gather(b):
            pltpu.async_copy(h_hbm.at[srcc[b]], rows[b], semg[b])

        def _wait_gather(b):
            pltpu.make_async_copy(h_hbm.at[srcc[b]], rows[b], semg[b]).wait()

        def _start_srcpa(ch, b):
            pltpu.async_copy(
                src_hbm.at[pl.ds(ebase + ch * C, C)], srcc[b], semsrc[b])
            pltpu.async_copy(
                pa_hbm.at[pl.ds(pbase + ch * C, C)], pac[b], sempa[b])

        def _wait_srcpa(ch, b):
            pltpu.make_async_copy(
                src_hbm.at[pl.ds(ebase + ch * C, C)], srcc[b], semsrc[b]).wait()
            pltpu.make_async_copy(
                pa_hbm.at[pl.ds(pbase + ch * C, C)], pac[b], sempa[b]).wait()

        def _start_dst(ch, b):
            pltpu.async_copy(
                dst_hbm.at[pl.ds(ebase + ch * C, C)], dstc[b], semd[b])

        def _wait_dst(ch, b):
            pltpu.make_async_copy(
                dst_hbm.at[pl.ds(ebase + ch * C, C)], dstc[b], semd[b]).wait()

        def _scale(ch, b):
            def _g(g, carry):
                f = pac[b][pl.ds(g * L, L)]
                for e in range(L):
                    s_vec = jnp.full((L,), f[e], jnp.float32)
                    row = g * L + e
                    for d in range(D // L):
                        rows[b][row, pl.ds(d * L, L)] = (
                            rows[b][row, pl.ds(d * L, L)] * s_vec)
                return carry
            lax.fori_loop(0, G, _g, 0)

        def _start_scatter(b):
            pltpu.async_copy(rows[b], acc.at[dstc[b]], sems[b], add=True)

        def _wait_scatter(b):
            pltpu.make_async_copy(rows[b], acc.at[dstc[b]], sems[b]).wait()

        def _clamp(ch):
            return jnp.minimum(ch, nch - 1)

        def _sub(ch, b):
            b2 = (b + 2) % 4
            b3 = (b + 3) % 4
            _wait_scatter(b2)
            _start_dst(_clamp(ch + 2), b2)
            _start_srcpa(_clamp(ch + 3), b3)
            _wait_srcpa(_clamp(ch + 2), b2)
            _start_gather(b2)
            _wait_gather(b)
            _scale(ch, b)
            _wait_dst(ch, b)
            _start_scatter(b)

        _start_srcpa(0, 0)
        _start_srcpa(1, 1)
        _start_srcpa(2, 2)
        _start_dst(0, 0)
        _start_dst(1, 1)
        _wait_srcpa(0, 0)
        _start_gather(0)
        _wait_srcpa(1, 1)
        _start_gather(1)
        _start_dst(2, 2)
        _start_srcpa(3, 3)
        _wait_srcpa(2, 2)
        _start_gather(2)
        _wait_gather(0)
        _scale(0, 0)
        _wait_dst(0, 0)
        _start_scatter(0)
        _start_dst(3, 3)
        _start_srcpa(4, 0)
        _wait_srcpa(3, 3)
        _start_gather(3)
        _wait_gather(1)
        _scale(1, 1)
        _wait_dst(1, 1)
        _start_scatter(1)

        NT = (nch - 2) // 4

        def _quad(t, carry):
            ch = 4 * t + 2
            _sub(ch, 2)
            _sub(ch + 1, 3)
            _sub(ch + 2, 0)
            _sub(ch + 3, 1)
            return carry
        lax.fori_loop(0, NT, _quad, 0)
        for ch in range(2 + 4 * NT, nch):
            _sub(ch, ch % 4)

        _wait_scatter((nch - 2) % 4)
        _wait_scatter((nch - 1) % 4)
        _wait_gather(nch % 4)
        _wait_gather((nch + 1) % 4)
        _wait_srcpa(nch - 1, (nch + 2) % 4)
        _wait_dst(nch - 1, nch % 4)
        _wait_dst(nch - 1, (nch + 1) % 4)

        plsc.subcore_barrier()

        def _dump(slab):
            pltpu.sync_copy(acc.at[pl.ds(slab * SR, SR)], rows[0])
            pltpu.sync_copy(
                rows[0], out_hbm.at[pl.ds(cid * N + slab * SR, SR)])
        _slabs(_dump)

    kw = dict(
        out_type=jax.ShapeDtypeStruct((2 * N, D), jnp.float32),
        mesh=mesh,
        compiler_params=pltpu.CompilerParams(needs_layout_passes=False),
        scratch_types=[
            [pltpu.VMEM((C,), jnp.int32) for _ in range(4)],
            [pltpu.VMEM((C,), jnp.float32) for _ in range(4)],
            [pltpu.VMEM((C,), jnp.int32) for _ in range(4)],
            [pltpu.VMEM((C, D), jnp.float32) for _ in range(4)],
            pltpu.VMEM_SHARED((N, D), jnp.float32),
            [pltpu.SemaphoreType.DMA for _ in range(4)],
            [pltpu.SemaphoreType.DMA for _ in range(4)],
            [pltpu.SemaphoreType.DMA for _ in range(4)],
            [pltpu.SemaphoreType.DMA for _ in range(4)],
            [pltpu.SemaphoreType.DMA for _ in range(4)],
        ],
    )

    if load_acc:
        @functools.partial(pl.kernel, **kw)
        def sck(h_hbm, pa_hbm, src_hbm, dst_hbm, pin_hbm, out_hbm,
                srcc, pac, dstc, rows, acc, semg, semsrc, sempa, semd, sems):
            _impl(h_hbm, pa_hbm, src_hbm, dst_hbm, pin_hbm, out_hbm,
                  srcc, pac, dstc, rows, acc, semg, semsrc, sempa, semd, sems)
    else:
        @functools.partial(pl.kernel, **kw)
        def sck(h_hbm, pa_hbm, src_hbm, dst_hbm, out_hbm,
                srcc, pac, dstc, rows, acc, semg, semsrc, sempa, semd, sems):
            _impl(h_hbm, pa_hbm, src_hbm, dst_hbm, None, out_hbm,
                  srcc, pac, dstc, rows, acc, semg, semsrc, sempa, semd, sems)
    return sck


def _pa_slice(rfT, prob_wT, E_k, block_off):
    RD = rfT.shape[0]
    BE = 6400
    return pl.pallas_call(
        _pa_body,
        grid=(E_k // BE,),
        in_specs=[
            pl.BlockSpec((RD, BE), lambda i: (0, i + block_off)),
            pl.BlockSpec((RD, 1), lambda i: (0, 0)),
        ],
        out_specs=pl.BlockSpec((1, BE), lambda i: (0, i)),
        out_shape=jax.ShapeDtypeStruct((1, E_k), jnp.float32),
    )(rfT, prob_wT)


def kernel(feat, cj, ci, review_feat, prob_w, edge_index):
    N, D = feat.shape
    E, RD = review_feat.shape

    NW, C = 32, 80
    NCH0 = 35
    E0 = NW * C * NCH0
    NCH1 = (E - E0) // (NW * C)
    E1 = E - E0
    BE = 6400

    rfT = review_feat.T
    wT = prob_w.T
    pa0 = _pa_slice(rfT, wT, E0, 0)
    pa1 = _pa_slice(rfT, wT, E1, E0 // BE)

    BN = 2000
    h = pl.pallas_call(
        _h_body,
        grid=(N // BN,),
        in_specs=[
            pl.BlockSpec((BN, D), lambda i: (i, 0)),
            pl.BlockSpec((BN, 1), lambda i: (i, 0)),
        ],
        out_specs=pl.BlockSpec((BN, D), lambda i: (i, 0)),
        out_shape=jax.ShapeDtypeStruct((N, D), jnp.float32),
    )(feat, cj)

    src = edge_index[0]
    dst = edge_index[1]
    sck0 = _make_sc_kernel(N, E, D, NCH0, 0, False)
    sck1 = _make_sc_kernel(N, E, D, NCH1, E0, True)
    partial0 = sck0(h, pa0.reshape(E0), src, dst)
    partial = sck1(h, pa1.reshape(E1), src, dst, partial0)

    NB = N // BN
    out = pl.pallas_call(
        _combine_body,
        grid=(NB,),
        in_specs=[
            pl.BlockSpec((BN, D), lambda i: (i, 0)),
            pl.BlockSpec((BN, D), lambda i: (i + NB, 0)),
            pl.BlockSpec((BN, 1), lambda i: (i, 0)),
        ],
        out_specs=pl.BlockSpec((BN, D), lambda i: (i, 0)),
        out_shape=jax.ShapeDtypeStruct((N, D), jnp.float32),
    )(partial, partial, ci)
    return out
